# Initial kernel scaffold; baseline (speedup 1.0000x reference)
import jax
import jax.numpy as jnp
from jax.experimental import pallas as pl


def kernel(x, table):
    raise NotImplementedError("write your pallas kernel here")


# speedup vs baseline: 2.3796x; 2.3796x over previous
import functools

import jax
import jax.numpy as jnp
from jax import lax
from jax.experimental import pallas as pl
from jax.experimental.pallas import tpu as pltpu
from jax.experimental.pallas import tpu_sc as plsc

B, L, K = 4096, 50, 8
VOCAB, D = 1000000, 32
N = B * L
IDX_PER_ROW = 128
POS_PER_IROW = IDX_PER_ROW // K
N_IDX_ROWS = (N * K) // IDX_PER_ROW

NW = 32
IROWS_PER_W = N_IDX_ROWS // NW
CHUNK_IROWS = 8
CHUNK_POS = CHUNK_IROWS * POS_PER_IROW
N_CHUNKS = IROWS_PER_W // CHUNK_IROWS
HALF = 16


def _make_sc_kernel():
  mesh = plsc.VectorSubcoreMesh(core_axis_name="c", subcore_axis_name="s")

  @functools.partial(
      pl.kernel,
      out_type=jax.ShapeDtypeStruct((N, D), jnp.float32),
      mesh=mesh,
      scratch_types=[
          pltpu.VMEM((CHUNK_IROWS, IDX_PER_ROW), jnp.int32),
          pltpu.VMEM((CHUNK_IROWS, IDX_PER_ROW, D), jnp.float32),
          pltpu.VMEM((CHUNK_POS, D), jnp.float32),
          pltpu.SemaphoreType.DMA,
      ],
      compiler_params=pltpu.CompilerParams(use_tc_tiling_on_sc=False),
  )
  def sc_embed(idx_hbm, table_hbm, out_hbm, idx_v, rows_v, out_v, sem):
    wid = lax.axis_index("s") * 2 + lax.axis_index("c")
    irow_base = wid * IROWS_PER_W
    pos_base = wid * (IROWS_PER_W * POS_PER_IROW)

    def chunk_body(c, carry):
      pltpu.sync_copy(
          idx_hbm.at[pl.ds(irow_base + c * CHUNK_IROWS, CHUNK_IROWS)], idx_v)
      copies = [
          pltpu.async_copy(table_hbm.at[idx_v.at[j]], rows_v.at[j], sem)
          for j in range(CHUNK_IROWS)
      ]
      for cp in copies:
        cp.wait()

      def pos_body(p, carry2):
        j = p // POS_PER_IROW
        q = (p % POS_PER_IROW) * K
        for h in range(0, D, HALF):
          acc = rows_v[j, q, pl.ds(h, HALF)]
          for k in range(1, K):
            acc = acc + rows_v[j, q + k, pl.ds(h, HALF)]
          out_v[p, pl.ds(h, HALF)] = acc
        return carry2

      lax.fori_loop(0, CHUNK_POS, pos_body, 0, unroll=2)

      pltpu.sync_copy(out_v, out_hbm.at[pl.ds(pos_base + c * CHUNK_POS,
                                              CHUNK_POS)])
      return carry

    lax.fori_loop(0, N_CHUNKS, chunk_body, 0)

  return sc_embed


_sc_embed = _make_sc_kernel()


@jax.jit
def kernel(x, table):
  idx = x.reshape(N_IDX_ROWS, IDX_PER_ROW).astype(jnp.int32)
  out = _sc_embed(idx, table)
  return out.reshape(B, L, D)


# speedup vs baseline: 2.5948x; 1.0905x over previous
import functools

import jax
import jax.numpy as jnp
from jax import lax
from jax.experimental import pallas as pl
from jax.experimental.pallas import tpu as pltpu
from jax.experimental.pallas import tpu_sc as plsc

B, L, K = 4096, 50, 8
VOCAB, D = 1000000, 32
N = B * L
IDX_PER_ROW = 128
POS_PER_IROW = IDX_PER_ROW // K
N_IDX_ROWS = (N * K) // IDX_PER_ROW

NW = 32
IROWS_PER_W = N_IDX_ROWS // NW
CHUNK_IROWS = 8
CHUNK_POS = CHUNK_IROWS * POS_PER_IROW
N_CHUNKS = IROWS_PER_W // CHUNK_IROWS
HALF = 16


def _make_sc_kernel():
  mesh = plsc.VectorSubcoreMesh(core_axis_name="c", subcore_axis_name="s")

  @functools.partial(
      pl.kernel,
      out_type=jax.ShapeDtypeStruct((N, D), jnp.float32),
      mesh=mesh,
      scratch_types=[
          pltpu.VMEM((2, CHUNK_IROWS, IDX_PER_ROW), jnp.int32),
          pltpu.VMEM((2, CHUNK_IROWS, IDX_PER_ROW, D), jnp.float32),
          pltpu.VMEM((2, CHUNK_POS, D), jnp.float32),
          pltpu.SemaphoreType.DMA,
          pltpu.SemaphoreType.DMA,
      ],
      compiler_params=pltpu.CompilerParams(use_tc_tiling_on_sc=False),
  )
  def sc_embed(idx_hbm, table_hbm, out_hbm, idx_v, rows_v, out_v, sem0, sem1):
    wid = lax.axis_index("s") * 2 + lax.axis_index("c")
    irow_base = wid * IROWS_PER_W
    pos_base = wid * (IROWS_PER_W * POS_PER_IROW)
    sems = (sem0, sem1)

    def stage_and_fire(c, buf):
      pltpu.sync_copy(
          idx_hbm.at[pl.ds(irow_base + c * CHUNK_IROWS, CHUNK_IROWS)],
          idx_v.at[buf])
      for j in range(CHUNK_IROWS):
        pltpu.async_copy(table_hbm.at[idx_v.at[buf, j]],
                         rows_v.at[buf, j], sems[buf])

    def drain(buf):
      for j in range(CHUNK_IROWS):
        pltpu.make_async_copy(table_hbm.at[idx_v.at[buf, j]],
                              rows_v.at[buf, j], sems[buf]).wait()

    def compute_and_store(c, buf):
      def pos_body(p, carry2):
        j = p // POS_PER_IROW
        q = (p % POS_PER_IROW) * K
        for h in range(0, D, HALF):
          acc = rows_v[buf, j, q, pl.ds(h, HALF)]
          for k in range(1, K):
            acc = acc + rows_v[buf, j, q + k, pl.ds(h, HALF)]
          out_v[buf, p, pl.ds(h, HALF)] = acc
        return carry2

      lax.fori_loop(0, CHUNK_POS, pos_body, 0, unroll=2)
      pltpu.sync_copy(out_v.at[buf],
                      out_hbm.at[pl.ds(pos_base + c * CHUNK_POS, CHUNK_POS)])

    stage_and_fire(0, 0)

    def pair_body(i, carry):
      c0 = i * 2
      stage_and_fire(c0 + 1, 1)
      drain(0)
      compute_and_store(c0, 0)
      @pl.when(c0 + 2 < N_CHUNKS)
      def _():
        stage_and_fire(c0 + 2, 0)
      drain(1)
      compute_and_store(c0 + 1, 1)
      return carry

    lax.fori_loop(0, N_CHUNKS // 2, pair_body, 0)

  return sc_embed


_sc_embed = _make_sc_kernel()


@jax.jit
def kernel(x, table):
  idx = x.reshape(N_IDX_ROWS, IDX_PER_ROW).astype(jnp.int32)
  out = _sc_embed(idx, table)
  return out.reshape(B, L, D)


# speedup vs baseline: 3.2405x; 1.2488x over previous
import functools

import jax
import jax.numpy as jnp
from jax import lax
from jax.experimental import pallas as pl
from jax.experimental.pallas import tpu as pltpu
from jax.experimental.pallas import tpu_sc as plsc

B, L, K = 4096, 50, 8
VOCAB, D = 1000000, 32
NW = 32
BPW = B // NW
NIDX = BPW * K
NGATH = NIDX // 128
HALF = 16
DT = D // 8


def _make_sc_kernel():
  mesh = plsc.VectorSubcoreMesh(core_axis_name="c", subcore_axis_name="s")

  @functools.partial(
      pl.kernel,
      out_type=jax.ShapeDtypeStruct((L, DT, NW, 8, 128), jnp.float32),
      mesh=mesh,
      scratch_types=[
          pltpu.VMEM((2, K, BPW), jnp.int32),
          pltpu.VMEM((2, K, BPW, D), jnp.float32),
          pltpu.VMEM((2, D, BPW), jnp.float32),
          pltpu.SemaphoreType.DMA,
          pltpu.SemaphoreType.DMA,
      ],
      compiler_params=pltpu.CompilerParams(use_tc_tiling_on_sc=False,
                                           needs_layout_passes=False),
  )
  def sc_embed(x_hbm, table_hbm, out_hbm, idx_v, rows_v, out_v, sem0, sem1):
    wid = lax.axis_index("s") * 2 + lax.axis_index("c")
    b0 = wid * BPW
    sems = (sem0, sem1)
    lane = lax.iota(jnp.int32, 16)

    def stage_and_fire(l, buf):
      pltpu.sync_copy(x_hbm.at[l, :, pl.ds(b0, BPW)], idx_v.at[buf])
      for k in range(K):
        pltpu.async_copy(table_hbm.at[idx_v.at[buf, k]],
                         rows_v.at[buf, k], sems[buf])

    def drain(buf):
      for k in range(K):
        pltpu.make_async_copy(table_hbm.at[idx_v.at[buf, k]],
                              rows_v.at[buf, k], sems[buf]).wait()

    def compute_and_store(l, buf):
      def pos_body(p, carry2):
        for h in range(0, D, HALF):
          acc = rows_v[buf, 0, p, pl.ds(h, HALF)]
          for k in range(1, K):
            acc = acc + rows_v[buf, k, p, pl.ds(h, HALF)]
          plsc.store_scatter(out_v.at[buf],
                             [lane + h, jnp.full((16,), p, jnp.int32)], acc)
        return carry2

      lax.fori_loop(0, BPW, pos_body, 0, unroll=2)
      for dt in range(DT):
        pltpu.sync_copy(out_v.at[buf, pl.ds(dt * 8, 8)],
                        out_hbm.at[l, dt, wid])

    stage_and_fire(0, 0)

    def pair_body(i, carry):
      l0 = i * 2
      stage_and_fire(l0 + 1, 1)
      drain(0)
      compute_and_store(l0, 0)
      @pl.when(l0 + 2 < L)
      def _():
        stage_and_fire(l0 + 2, 0)
      drain(1)
      compute_and_store(l0 + 1, 1)
      return carry

    lax.fori_loop(0, L // 2, pair_body, 0)

  return sc_embed


_sc_embed = _make_sc_kernel()


@jax.jit
def kernel(x, table):
  out5 = _sc_embed(jnp.transpose(x, (1, 2, 0)), table)
  return out5.transpose((2, 4, 0, 1, 3)).reshape(B, L, D)
